# baseline (speedup 1.0000x reference)
import jax
import jax.numpy as jnp
from jax.experimental import pallas as pl
from jax.experimental.pallas import tpu as pltpu

N_TOKENS = 8192
HIDDEN = 2048
PRED_H = 256
N_EXPERTS = 64
N_CAND = 16
BT = 512
NB = N_TOKENS // BT


def _tdot(a, b):
    return jax.lax.dot_general(a, b, (((1,), (1,)), ((), ())),
                               preferred_element_type=jnp.float32)


def _router_body(x_ref, w1_ref, b1_ref, w2_ref, b2_ref, eb_ref, rw_ref,
                 w_out_ref, id_out_ref, p_scr, f_scr):
    i = pl.program_id(0)

    tslot = jax.lax.rem(i + 1, 2)
    neg = jnp.float32(-1e9)
    cur = p_scr[tslot]
    for _ in range(N_CAND - 1):
        m = jnp.max(cur, axis=1, keepdims=True)
        cur = jnp.where(cur >= m, -jnp.inf, cur)
    thresh = jnp.max(cur, axis=1, keepdims=True)

    f = f_scr[tslot]
    cand = jnp.logical_or(cur == -jnp.inf, cur >= thresh)
    g = jnp.where(cand, f, neg)
    iota = jax.lax.broadcasted_iota(jnp.int32, g.shape, 1)
    iota_f = iota.astype(jnp.float32)
    big = jnp.float32(N_EXPERTS)
    v1 = jnp.max(g, axis=1, keepdims=True)
    i1f = jnp.min(jnp.where(g >= v1, iota_f, big), axis=1, keepdims=True)
    g2 = jnp.where(iota_f == i1f, neg, g)
    v2 = jnp.max(g2, axis=1, keepdims=True)
    i2f = jnp.min(jnp.where(g2 >= v2, iota_f, big), axis=1, keepdims=True)

    e = jnp.exp(v2 - v1)
    inv = 1.0 / (1.0 + e)
    zf = jnp.zeros_like(g)
    zi = jnp.zeros_like(iota)
    w_out_ref[...] = jnp.where(iota == 0, inv,
                               jnp.where(iota == 1, e * inv, zf))
    id_out_ref[...] = jnp.where(iota == 0, i1f.astype(jnp.int32),
                                jnp.where(iota == 1, i2f.astype(jnp.int32),
                                          zi))

    x = x_ref[...]
    h = jnp.maximum(_tdot(x, w1_ref[...]) + b1_ref[...], 0.0)
    slot = jax.lax.rem(i, 2)
    p_scr[slot] = _tdot(h, w2_ref[...]) + (b2_ref[...] + eb_ref[...])
    f_scr[slot] = _tdot(x, rw_ref[...])


def kernel(x, fc1_w, fc1_b, fc2_w, fc2_b, expert_bias, router_w):
    b1 = fc1_b.reshape(1, PRED_H)
    b2 = fc2_b.reshape(1, N_EXPERTS)
    eb = expert_bias.reshape(1, N_EXPERTS)

    out_w, out_id = pl.pallas_call(
        _router_body,
        grid=(NB + 1,),
        in_specs=[
            pl.BlockSpec((BT, HIDDEN), lambda i: (jnp.minimum(i, NB - 1), 0)),
            pl.BlockSpec((PRED_H, HIDDEN), lambda i: (0, 0)),
            pl.BlockSpec((1, PRED_H), lambda i: (0, 0)),
            pl.BlockSpec((N_EXPERTS, PRED_H), lambda i: (0, 0)),
            pl.BlockSpec((1, N_EXPERTS), lambda i: (0, 0)),
            pl.BlockSpec((1, N_EXPERTS), lambda i: (0, 0)),
            pl.BlockSpec((N_EXPERTS, HIDDEN), lambda i: (0, 0)),
        ],
        out_specs=[
            pl.BlockSpec((BT, N_EXPERTS), lambda i: (jnp.maximum(i - 1, 0), 0)),
            pl.BlockSpec((BT, N_EXPERTS), lambda i: (jnp.maximum(i - 1, 0), 0)),
        ],
        out_shape=[
            jax.ShapeDtypeStruct((N_TOKENS, N_EXPERTS), jnp.float32),
            jax.ShapeDtypeStruct((N_TOKENS, N_EXPERTS), jnp.int32),
        ],
        scratch_shapes=[
            pltpu.VMEM((2, BT, N_EXPERTS), jnp.float32),
            pltpu.VMEM((2, BT, N_EXPERTS), jnp.float32),
        ],
    )(x, fc1_w, b1, fc2_w, b2, eb, router_w)
    return out_w[:, :2], out_id[:, :2]
